# baseline (speedup 1.0000x reference)
import functools

import jax
import jax.numpy as jnp
from jax import lax
from jax.experimental import pallas as pl
from jax.experimental.pallas import tpu as pltpu
from jax.experimental.pallas import tpu_sc as plsc

BATCH = 16384
EMBED_DIM = 32
NUM_CORES = 2
NUM_SUBCORES = 16
NUM_WORKERS = NUM_CORES * NUM_SUBCORES
B_PER_W = BATCH // NUM_WORKERS
GROUP = 16
WINDOW = 64
N_WINDOWS = B_PER_W // WINDOW

_MESH = plsc.VectorSubcoreMesh(core_axis_name="c", subcore_axis_name="s")


@functools.partial(
    pl.kernel,
    mesh=_MESH,
    out_type=jax.ShapeDtypeStruct((BATCH, EMBED_DIM), jnp.float32),
    scratch_types=[
        pltpu.VMEM((B_PER_W,), jnp.int32),
        pltpu.VMEM((B_PER_W, EMBED_DIM), jnp.float32),
        pltpu.SemaphoreType.DMA,
        pltpu.SemaphoreType.DMA,
        pltpu.SemaphoreType.DMA,
        pltpu.SemaphoreType.DMA,
    ],
    compiler_params=pltpu.CompilerParams(needs_layout_passes=False),
)
def _sc_gather(idx_hbm, table_hbm, out_hbm, idx_v, rows_v, s0, s1, s2, s3):
    wid = lax.axis_index("s") * NUM_CORES + lax.axis_index("c")
    base = wid * B_PER_W
    sems = (s0, s1, s2, s3)
    pltpu.sync_copy(idx_hbm.at[wid], idx_v)

    @pl.loop(0, N_WINDOWS)
    def _win(g):
        copies = []
        for q in range(WINDOW // GROUP):
            iv = idx_v[pl.ds(g * WINDOW + q * GROUP, GROUP)]
            for t in range(GROUP):
                i = g * WINDOW + q * GROUP + t
                copies.append(
                    pltpu.async_copy(
                        table_hbm.at[iv[t]],
                        out_hbm.at[base + i],
                        sems[q],
                    )
                )
        for c in copies:
            c.wait()


def kernel(topic_labels, embedding_weight):
    idx = topic_labels.astype(jnp.int32).reshape(NUM_WORKERS, B_PER_W)
    return _sc_gather(idx, embedding_weight)
